# baseline (speedup 1.0000x reference)
import functools

import jax
import jax.numpy as jnp
from jax.experimental import pallas as pl
from jax.experimental.pallas import tpu as pltpu

N = 10000
E = 320000
D = 128
H = 128
L = 3

ET = 2000
NT = 2000


def _silu(v):
    return v * jax.nn.sigmoid(v)


def _edge_body(g_ref, aux_ref, wr_ref, b1_ref, w2_ref, b2_ref,
               c1_ref, cb1_ref, c2_ref, cb2_ref, m_ref, t_ref):
    g = g_ref[...]
    aux = aux_ref[...]
    r2 = aux[:, 3:4]
    m1 = _silu(g + r2 * wr_ref[...] + b1_ref[...])
    m2 = _silu(jnp.dot(m1, w2_ref[...], preferred_element_type=jnp.float32)
               + b2_ref[...])
    u = _silu(jnp.dot(m2, c1_ref[...], preferred_element_type=jnp.float32)
              + cb1_ref[...])
    w = jnp.dot(u, c2_ref[...], preferred_element_type=jnp.float32) + cb2_ref[...]
    m_ref[...] = m2
    t_ref[...] = aux * w


def _edge_pipeline(G, aux, wr, b1, W2, b2, C1, cb1, C2, cb2):
    grid = (E // ET,)
    full = lambda shape: pl.BlockSpec(shape, lambda i: (0, 0))
    return pl.pallas_call(
        _edge_body,
        grid=grid,
        in_specs=[
            pl.BlockSpec((ET, H), lambda i: (i, 0)),
            pl.BlockSpec((ET, 4), lambda i: (i, 0)),
            full((1, H)), full((1, H)), full((H, H)), full((1, H)),
            full((H, H)), full((1, H)), full((H, 1)), full((1, 1)),
        ],
        out_specs=[
            pl.BlockSpec((ET, H), lambda i: (i, 0)),
            pl.BlockSpec((ET, 4), lambda i: (i, 0)),
        ],
        out_shape=[
            jax.ShapeDtypeStruct((E, H), jnp.float32),
            jax.ShapeDtypeStruct((E, 4), jnp.float32),
        ],
    )(G, aux, wr, b1, W2, b2, C1, cb1, C2, cb2)


def _node_body(h_ref, agg_ref, xa_ref, wa_ref, wb_ref, nb1_ref,
               w2_ref, nb2_ref, h_out_ref, x_out_ref):
    h = h_ref[...]
    agg = agg_ref[...]
    hn = _silu(jnp.dot(h, wa_ref[...], preferred_element_type=jnp.float32)
               + jnp.dot(agg, wb_ref[...], preferred_element_type=jnp.float32)
               + nb1_ref[...])
    hn = jnp.dot(hn, w2_ref[...], preferred_element_type=jnp.float32) + nb2_ref[...]
    h_out_ref[...] = h + hn
    xa = xa_ref[...]
    cnt = jnp.maximum(xa[:, 7:8], 1.0)
    x_out_ref[...] = xa[:, 0:4] + xa[:, 4:8] / cnt


def _node_pipeline(h, agg_m, xa, Wa, Wb, nb1, W2, nb2):
    grid = (N // NT,)
    full = lambda shape: pl.BlockSpec(shape, lambda i: (0, 0))
    return pl.pallas_call(
        _node_body,
        grid=grid,
        in_specs=[
            pl.BlockSpec((NT, H), lambda i: (i, 0)),
            pl.BlockSpec((NT, H), lambda i: (i, 0)),
            pl.BlockSpec((NT, 8), lambda i: (i, 0)),
            full((H, H)), full((H, H)), full((1, H)), full((H, D)), full((1, D)),
        ],
        out_specs=[
            pl.BlockSpec((NT, D), lambda i: (i, 0)),
            pl.BlockSpec((NT, 4), lambda i: (i, 0)),
        ],
        out_shape=[
            jax.ShapeDtypeStruct((N, D), jnp.float32),
            jax.ShapeDtypeStruct((N, 4), jnp.float32),
        ],
    )(h, agg_m, xa, Wa, Wb, nb1, W2, nb2)


def _proj_body(h_ref, w_ref, a_ref, b_ref):
    h = h_ref[...]
    a_ref[...] = jnp.dot(h, w_ref[0:D, :], preferred_element_type=jnp.float32)
    b_ref[...] = jnp.dot(h, w_ref[D:2 * D, :], preferred_element_type=jnp.float32)


def _proj_pipeline(h, W1d):
    grid = (N // NT,)
    return pl.pallas_call(
        _proj_body,
        grid=grid,
        in_specs=[
            pl.BlockSpec((NT, D), lambda i: (i, 0)),
            pl.BlockSpec((2 * D, H), lambda i: (0, 0)),
        ],
        out_specs=[
            pl.BlockSpec((NT, H), lambda i: (i, 0)),
            pl.BlockSpec((NT, H), lambda i: (i, 0)),
        ],
        out_shape=[
            jax.ShapeDtypeStruct((N, H), jnp.float32),
            jax.ShapeDtypeStruct((N, H), jnp.float32),
        ],
    )(h, W1d)


def _fc_body(h_ref, w_ref, b_ref, y_ref):
    y_ref[...] = jnp.dot(h_ref[...], w_ref[...],
                         preferred_element_type=jnp.float32) + b_ref[...]


def _fc_pipeline(h, fc_W, fc_b):
    grid = (N // NT,)
    return pl.pallas_call(
        _fc_body,
        grid=grid,
        in_specs=[
            pl.BlockSpec((NT, D), lambda i: (i, 0)),
            pl.BlockSpec((D, 128), lambda i: (0, 0)),
            pl.BlockSpec((1, 128), lambda i: (0, 0)),
        ],
        out_specs=pl.BlockSpec((NT, 128), lambda i: (i, 0)),
        out_shape=jax.ShapeDtypeStruct((N, 128), jnp.float32),
    )(h, fc_W, fc_b[None, :])


def kernel(h, x, edge_index, edge_W1, edge_b1, edge_W2, edge_b2,
           coord_W1, coord_b1, coord_W2, coord_b2,
           node_W1, node_b1, node_W2, node_b2, fc_W, fc_b):
    src = edge_index[0]
    dst = edge_index[1]

    cnt = jax.ops.segment_sum(jnp.ones((E,), jnp.float32), dst, num_segments=N)

    xp = jnp.pad(x, ((0, 0), (0, 1)))

    for l in range(L):
        A, B = _proj_pipeline(h, edge_W1[l])
        wr = edge_W1[l, 2 * D:2 * D + 1, :]

        G = A[dst] + B[src]
        diff4 = xp[dst] - xp[src]
        r2 = jnp.sum(diff4 * diff4, axis=-1, keepdims=True)
        aux = jnp.concatenate([diff4[:, 0:3], r2], axis=1)

        m, t = _edge_pipeline(
            G, aux, wr, edge_b1[l][None, :], edge_W2[l], edge_b2[l][None, :],
            coord_W1[l], coord_b1[l][None, :], coord_W2[l],
            coord_b2[l][None, :])

        agg_m = jax.ops.segment_sum(m, dst, num_segments=N)
        agg_x = jax.ops.segment_sum(t[:, 0:3], dst, num_segments=N)

        xa = jnp.concatenate(
            [xp, agg_x, cnt[:, None]], axis=1)

        h, xp = _node_pipeline(
            h, agg_m, xa,
            node_W1[l, 0:D, :], node_W1[l, D:, :], node_b1[l][None, :],
            node_W2[l], node_b2[l][None, :])

    return _fc_pipeline(h, fc_W, fc_b)


# speedup vs baseline: 2.0976x; 2.0976x over previous
import functools

import jax
import jax.numpy as jnp
from jax import lax
from jax.experimental import pallas as pl
from jax.experimental.pallas import tpu as pltpu
from jax.experimental.pallas import tpu_sc as plsc

N = 10000
E = 320000
D = 128
H = 128
L = 3

ET = 2000
NT = 2000

NC = 2
NS = 16
NW = NC * NS
CH = 128
ROWS = E // CH
RPW = ROWS // NW
REM = ROWS - RPW * NW
NP = 10240


def _silu(v):
    return v * jax.nn.sigmoid(v)


def _sc_gather(AX, BX, dst2, src2):
    mesh = plsc.VectorSubcoreMesh(core_axis_name="c", subcore_axis_name="s")

    @functools.partial(
        pl.kernel,
        out_type=[
            jax.ShapeDtypeStruct((E, 2 * H), jnp.float32),
            jax.ShapeDtypeStruct((E, 2 * H), jnp.float32),
        ],
        mesh=mesh,
        scratch_types=[
            pltpu.VMEM((CH,), jnp.int32),
            pltpu.VMEM((CH,), jnp.int32),
            pltpu.VMEM((CH, 2 * H), jnp.float32),
            pltpu.VMEM((CH, 2 * H), jnp.float32),
            pltpu.SemaphoreType.DMA,
        ],
    )
    def k(ax_hbm, bx_hbm, d_hbm, s_hbm, ga_out, gb_out,
          di_v, si_v, ga_v, gb_v, sem):
        wid = lax.axis_index("s") * NC + lax.axis_index("c")
        base = wid * RPW + jnp.minimum(wid, REM)
        count = RPW + jnp.where(wid < REM, 1, 0)

        def body(r, carry):
            pltpu.sync_copy(d_hbm.at[r], di_v)
            pltpu.sync_copy(s_hbm.at[r], si_v)
            c1 = pltpu.async_copy(ax_hbm.at[di_v], ga_v, sem)
            c2 = pltpu.async_copy(bx_hbm.at[si_v], gb_v, sem)
            c1.wait()
            c2.wait()
            e0 = r * CH
            pltpu.sync_copy(ga_v, ga_out.at[pl.ds(e0, CH)])
            pltpu.sync_copy(gb_v, gb_out.at[pl.ds(e0, CH)])
            return carry

        lax.fori_loop(base, base + count, body, 0)

    return k(AX, BX, dst2, src2)


def _sc_scatter(m, t, dst2):
    mesh = plsc.VectorSubcoreMesh(core_axis_name="c", subcore_axis_name="s")
    ZR = NP // NS
    ZC = ZR // CH

    @functools.partial(
        pl.kernel,
        out_type=[
            jax.ShapeDtypeStruct((NC, NP, H), jnp.float32),
            jax.ShapeDtypeStruct((NC, NP, 16), jnp.float32),
        ],
        mesh=mesh,
        scratch_types=[
            pltpu.VMEM((CH,), jnp.int32),
            pltpu.VMEM((CH, H), jnp.float32),
            pltpu.VMEM((CH, 16), jnp.float32),
            pltpu.VMEM_SHARED((NP, H), jnp.float32),
            pltpu.VMEM_SHARED((NP, 16), jnp.float32),
            pltpu.SemaphoreType.DMA,
        ],
    )
    def k(m_hbm, t_hbm, d_hbm, am_out, ax_out,
          di_v, mv, tv, accm, accx, sem):
        cid = lax.axis_index("c")
        sid = lax.axis_index("s")
        wid = sid * NC + cid

        def zrow(j, carry):
            for kk in range(H // 16):
                mv[j, pl.ds(kk * 16, 16)] = jnp.zeros((16,), jnp.float32)
            tv[j, pl.ds(0, 16)] = jnp.zeros((16,), jnp.float32)
            return carry
        lax.fori_loop(0, CH, zrow, 0)
        for z in range(ZC):
            r0 = sid * ZR + z * CH
            pltpu.sync_copy(mv, accm.at[pl.ds(r0, CH)])
            pltpu.sync_copy(tv, accx.at[pl.ds(r0, CH)])
        plsc.subcore_barrier()

        base = wid * RPW + jnp.minimum(wid, REM)
        count = RPW + jnp.where(wid < REM, 1, 0)

        def body(r, carry):
            e0 = r * CH
            pltpu.sync_copy(d_hbm.at[r], di_v)
            c1 = pltpu.async_copy(m_hbm.at[pl.ds(e0, CH)], mv, sem)
            c2 = pltpu.async_copy(t_hbm.at[pl.ds(e0, CH)], tv, sem)
            c1.wait()
            c2.wait()
            pltpu.sync_copy(mv, accm.at[di_v], add=True)
            pltpu.sync_copy(tv, accx.at[di_v], add=True)
            return carry

        lax.fori_loop(base, base + count, body, 0)
        plsc.subcore_barrier()

        for z in range(ZC):
            r0 = sid * ZR + z * CH
            pltpu.sync_copy(accm.at[pl.ds(r0, CH)], mv)
            pltpu.sync_copy(mv, am_out.at[cid, pl.ds(r0, CH)])
            pltpu.sync_copy(accx.at[pl.ds(r0, CH)], tv)
            pltpu.sync_copy(tv, ax_out.at[cid, pl.ds(r0, CH)])

    return k(m, t, dst2)


def _edge_body(gxa_ref, gxb_ref, wr_ref, b1_ref, w2_ref, b2_ref,
               c1_ref, cb1_ref, c2_ref, cb2_ref, m_ref, t_ref):
    gxa = gxa_ref[...]
    gxb = gxb_ref[...]
    g = gxa[:, 0:H] + gxb[:, 0:H]
    diff = gxa[:, H:H + 16] - gxb[:, H:H + 16]
    r2 = jnp.sum(diff * diff, axis=1, keepdims=True)
    m1 = _silu(g + r2 * wr_ref[...] + b1_ref[...])
    m2 = _silu(jnp.dot(m1, w2_ref[...], preferred_element_type=jnp.float32)
               + b2_ref[...])
    u = _silu(jnp.dot(m2, c1_ref[...], preferred_element_type=jnp.float32)
              + cb1_ref[...])
    w = jnp.dot(u, c2_ref[...], preferred_element_type=jnp.float32) + cb2_ref[...]
    m_ref[...] = m2
    ii = lax.broadcasted_iota(jnp.int32, (1, 16), 1)
    t_ref[...] = jnp.where(ii == 3, 1.0, diff * w)


def _edge_pipeline(GXA, GXB, wr, b1, W2, b2, C1, cb1, C2, cb2):
    grid = (E // ET,)
    full = lambda shape: pl.BlockSpec(shape, lambda i: (0, 0))
    return pl.pallas_call(
        _edge_body,
        grid=grid,
        in_specs=[
            pl.BlockSpec((ET, 2 * H), lambda i: (i, 0)),
            pl.BlockSpec((ET, 2 * H), lambda i: (i, 0)),
            full((1, H)), full((1, H)), full((H, H)), full((1, H)),
            full((H, H)), full((1, H)), full((H, 1)), full((1, 1)),
        ],
        out_specs=[
            pl.BlockSpec((ET, H), lambda i: (i, 0)),
            pl.BlockSpec((ET, 16), lambda i: (i, 0)),
        ],
        out_shape=[
            jax.ShapeDtypeStruct((E, H), jnp.float32),
            jax.ShapeDtypeStruct((E, 16), jnp.float32),
        ],
    )(GXA, GXB, wr, b1, W2, b2, C1, cb1, C2, cb2)


def _node_body(h_ref, am0_ref, am1_ref, ax0_ref, ax1_ref, xp_ref,
               wa_ref, wb_ref, nb1_ref, w2_ref, nb2_ref,
               h_out_ref, x_out_ref):
    h = h_ref[...]
    agg = am0_ref[...] + am1_ref[...]
    hn = _silu(jnp.dot(h, wa_ref[...], preferred_element_type=jnp.float32)
               + jnp.dot(agg, wb_ref[...], preferred_element_type=jnp.float32)
               + nb1_ref[...])
    hn = jnp.dot(hn, w2_ref[...], preferred_element_type=jnp.float32) + nb2_ref[...]
    h_out_ref[...] = h + hn
    axs = ax0_ref[...] + ax1_ref[...]
    cnt = jnp.maximum(axs[:, 3:4], 1.0)
    xup = jnp.concatenate(
        [axs / cnt, jnp.zeros((axs.shape[0], H - 16), jnp.float32)], axis=1)
    x_out_ref[...] = xp_ref[...] + xup


def _node_pipeline(h, am0, am1, ax0, ax1, xp, Wa, Wb, nb1, W2, nb2):
    grid = (N // NT,)
    full = lambda shape: pl.BlockSpec(shape, lambda i: (0, 0))
    return pl.pallas_call(
        _node_body,
        grid=grid,
        in_specs=[
            pl.BlockSpec((NT, H), lambda i: (i, 0)),
            pl.BlockSpec((NT, H), lambda i: (i, 0)),
            pl.BlockSpec((NT, H), lambda i: (i, 0)),
            pl.BlockSpec((NT, 16), lambda i: (i, 0)),
            pl.BlockSpec((NT, 16), lambda i: (i, 0)),
            pl.BlockSpec((NT, H), lambda i: (i, 0)),
            full((H, H)), full((H, H)), full((1, H)), full((H, D)), full((1, D)),
        ],
        out_specs=[
            pl.BlockSpec((NT, D), lambda i: (i, 0)),
            pl.BlockSpec((NT, H), lambda i: (i, 0)),
        ],
        out_shape=[
            jax.ShapeDtypeStruct((N, D), jnp.float32),
            jax.ShapeDtypeStruct((N, H), jnp.float32),
        ],
    )(h, am0, am1, ax0, ax1, xp, Wa, Wb, nb1, W2, nb2)


def _proj_body(h_ref, xp_ref, w_ref, ax_ref, bx_ref):
    h = h_ref[...]
    xp = xp_ref[...]
    ax_ref[:, 0:H] = jnp.dot(h, w_ref[0:D, :], preferred_element_type=jnp.float32)
    ax_ref[:, H:2 * H] = xp
    bx_ref[:, 0:H] = jnp.dot(h, w_ref[D:2 * D, :],
                             preferred_element_type=jnp.float32)
    bx_ref[:, H:2 * H] = xp


def _proj_pipeline(h, xp, W1d):
    grid = (N // NT,)
    return pl.pallas_call(
        _proj_body,
        grid=grid,
        in_specs=[
            pl.BlockSpec((NT, D), lambda i: (i, 0)),
            pl.BlockSpec((NT, H), lambda i: (i, 0)),
            pl.BlockSpec((2 * D, H), lambda i: (0, 0)),
        ],
        out_specs=[
            pl.BlockSpec((NT, 2 * H), lambda i: (i, 0)),
            pl.BlockSpec((NT, 2 * H), lambda i: (i, 0)),
        ],
        out_shape=[
            jax.ShapeDtypeStruct((N, 2 * H), jnp.float32),
            jax.ShapeDtypeStruct((N, 2 * H), jnp.float32),
        ],
    )(h, xp, W1d)


def _fc_body(h_ref, w_ref, b_ref, y_ref):
    y_ref[...] = jnp.dot(h_ref[...], w_ref[...],
                         preferred_element_type=jnp.float32) + b_ref[...]


def _fc_pipeline(h, fc_W, fc_b):
    grid = (N // NT,)
    return pl.pallas_call(
        _fc_body,
        grid=grid,
        in_specs=[
            pl.BlockSpec((NT, D), lambda i: (i, 0)),
            pl.BlockSpec((D, 128), lambda i: (0, 0)),
            pl.BlockSpec((1, 128), lambda i: (0, 0)),
        ],
        out_specs=pl.BlockSpec((NT, 128), lambda i: (i, 0)),
        out_shape=jax.ShapeDtypeStruct((N, 128), jnp.float32),
    )(h, fc_W, fc_b[None, :])


def kernel(h, x, edge_index, edge_W1, edge_b1, edge_W2, edge_b2,
           coord_W1, coord_b1, coord_W2, coord_b2,
           node_W1, node_b1, node_W2, node_b2, fc_W, fc_b):
    src2 = edge_index[0].reshape(ROWS, CH)
    dst2 = edge_index[1].reshape(ROWS, CH)

    xp = jnp.pad(x, ((0, 0), (0, H - 3)))

    for l in range(L):
        AX, BX = _proj_pipeline(h, xp, edge_W1[l])
        wr = edge_W1[l, 2 * D:2 * D + 1, :]

        GXA, GXB = _sc_gather(AX, BX, dst2, src2)

        m, t = _edge_pipeline(
            GXA, GXB, wr, edge_b1[l][None, :], edge_W2[l],
            edge_b2[l][None, :], coord_W1[l], coord_b1[l][None, :],
            coord_W2[l], coord_b2[l][None, :])

        dst = edge_index[1]
        am0 = jax.ops.segment_sum(m, dst, num_segments=N)
        ax0 = jax.ops.segment_sum(t, dst, num_segments=N)
        am1 = jnp.zeros_like(am0)
        ax1 = jnp.zeros_like(ax0)

        h, xp = _node_pipeline(
            h, am0, am1, ax0, ax1, xp,
            node_W1[l, 0:D, :], node_W1[l, D:, :], node_b1[l][None, :],
            node_W2[l], node_b2[l][None, :])

    return _fc_pipeline(h, fc_W, fc_b)


# speedup vs baseline: 2.6240x; 1.2510x over previous
import functools

import jax
import jax.numpy as jnp
from jax import lax
from jax.experimental import pallas as pl
from jax.experimental.pallas import tpu as pltpu
from jax.experimental.pallas import tpu_sc as plsc

N = 10000
E = 320000
D = 128
H = 128
L = 3

ET = 2000
NT = 2000

NC = 2
NS = 16
NW = NC * NS
CH = 128
ROWS = E // CH
RPW = ROWS // NW
REM = ROWS - RPW * NW
NP = 10240


def _silu(v):
    return v * jax.nn.sigmoid(v)


def _sc_gather(AX, BX, dst2, src2):
    mesh = plsc.VectorSubcoreMesh(core_axis_name="c", subcore_axis_name="s")

    @functools.partial(
        pl.kernel,
        out_type=[
            jax.ShapeDtypeStruct((E, 2 * H), jnp.float32),
            jax.ShapeDtypeStruct((E, 2 * H), jnp.float32),
        ],
        mesh=mesh,
        scratch_types=[
            pltpu.VMEM((CH,), jnp.int32),
            pltpu.VMEM((CH,), jnp.int32),
            pltpu.VMEM((CH, 2 * H), jnp.float32),
            pltpu.VMEM((CH, 2 * H), jnp.float32),
            pltpu.SemaphoreType.DMA,
        ],
    )
    def k(ax_hbm, bx_hbm, d_hbm, s_hbm, ga_out, gb_out,
          di_v, si_v, ga_v, gb_v, sem):
        wid = lax.axis_index("s") * NC + lax.axis_index("c")
        base = wid * RPW + jnp.minimum(wid, REM)
        count = RPW + jnp.where(wid < REM, 1, 0)

        def body(r, carry):
            pltpu.sync_copy(d_hbm.at[r], di_v)
            pltpu.sync_copy(s_hbm.at[r], si_v)
            c1 = pltpu.async_copy(ax_hbm.at[di_v], ga_v, sem)
            c2 = pltpu.async_copy(bx_hbm.at[si_v], gb_v, sem)
            c1.wait()
            c2.wait()
            e0 = r * CH
            pltpu.sync_copy(ga_v, ga_out.at[pl.ds(e0, CH)])
            pltpu.sync_copy(gb_v, gb_out.at[pl.ds(e0, CH)])
            return carry

        lax.fori_loop(base, base + count, body, 0)

    return k(AX, BX, dst2, src2)


def _sc_scatter(m, dst2):
    mesh = plsc.VectorSubcoreMesh(core_axis_name="c", subcore_axis_name="s")
    ZR = NP // NS
    ZC = ZR // CH

    @functools.partial(
        pl.kernel,
        out_type=[
            jax.ShapeDtypeStruct((NC, NP, H), jnp.float32),
        ],
        mesh=mesh,
        scratch_types=[
            pltpu.VMEM((CH,), jnp.int32),
            pltpu.VMEM((CH, H), jnp.float32),
            pltpu.VMEM_SHARED((NP, H), jnp.float32),
            pltpu.SemaphoreType.DMA,
        ],
    )
    def k(m_hbm, d_hbm, am_out,
          di_v, mv, accm, sem):
        cid = lax.axis_index("c")
        sid = lax.axis_index("s")
        wid = sid * NC + cid

        def zrow(j, carry):
            for kk in range(H // 16):
                mv[j, pl.ds(kk * 16, 16)] = jnp.zeros((16,), jnp.float32)
            return carry
        lax.fori_loop(0, CH, zrow, 0)
        for z in range(ZC):
            r0 = sid * ZR + z * CH
            pltpu.sync_copy(mv, accm.at[pl.ds(r0, CH)])
        plsc.subcore_barrier()

        base = wid * RPW + jnp.minimum(wid, REM)
        count = RPW + jnp.where(wid < REM, 1, 0)

        def body(r, carry):
            e0 = r * CH
            pltpu.sync_copy(d_hbm.at[r], di_v)
            c1 = pltpu.async_copy(m_hbm.at[pl.ds(e0, CH)], mv, sem)
            c1.wait()
            pltpu.sync_copy(mv, accm.at[di_v], add=True)
            return carry

        lax.fori_loop(base, base + count, body, 0)
        plsc.subcore_barrier()

        for z in range(ZC):
            r0 = sid * ZR + z * CH
            pltpu.sync_copy(accm.at[pl.ds(r0, CH)], mv)
            pltpu.sync_copy(mv, am_out.at[cid, pl.ds(r0, CH)])

    return k(m, dst2)


def _edge_body(gxa_ref, gxb_ref, wr_ref, b1_ref, w2_ref, b2_ref,
               c1_ref, cb1_ref, c2_ref, cb2_ref, m_ref, t_ref):
    gxa = gxa_ref[...]
    gxb = gxb_ref[...]
    g = gxa[:, 0:H] + gxb[:, 0:H]
    diff = gxa[:, H:H + 16] - gxb[:, H:H + 16]
    r2 = jnp.sum(diff * diff, axis=1, keepdims=True)
    m1 = _silu(g + r2 * wr_ref[...] + b1_ref[...])
    m2 = _silu(jnp.dot(m1, w2_ref[...], preferred_element_type=jnp.float32)
               + b2_ref[...])
    u = _silu(jnp.dot(m2, c1_ref[...], preferred_element_type=jnp.float32)
              + cb1_ref[...])
    w = jnp.dot(u, c2_ref[...], preferred_element_type=jnp.float32) + cb2_ref[...]
    m_ref[...] = m2
    ii = lax.broadcasted_iota(jnp.int32, (1, 16), 1)
    t_ref[...] = jnp.where(ii == 3, 1.0, diff * w)


def _edge_pipeline(GXA, GXB, wr, b1, W2, b2, C1, cb1, C2, cb2):
    grid = (E // ET,)
    full = lambda shape: pl.BlockSpec(shape, lambda i: (0, 0))
    return pl.pallas_call(
        _edge_body,
        grid=grid,
        in_specs=[
            pl.BlockSpec((ET, 2 * H), lambda i: (i, 0)),
            pl.BlockSpec((ET, 2 * H), lambda i: (i, 0)),
            full((1, H)), full((1, H)), full((H, H)), full((1, H)),
            full((H, H)), full((1, H)), full((H, 1)), full((1, 1)),
        ],
        out_specs=[
            pl.BlockSpec((ET, H), lambda i: (i, 0)),
            pl.BlockSpec((ET, 16), lambda i: (i, 0)),
        ],
        out_shape=[
            jax.ShapeDtypeStruct((E, H), jnp.float32),
            jax.ShapeDtypeStruct((E, 16), jnp.float32),
        ],
    )(GXA, GXB, wr, b1, W2, b2, C1, cb1, C2, cb2)


def _node_body(h_ref, am0_ref, am1_ref, ax0_ref, ax1_ref, xp_ref,
               wa_ref, wb_ref, nb1_ref, w2_ref, nb2_ref,
               h_out_ref, x_out_ref):
    h = h_ref[...]
    agg = am0_ref[...] + am1_ref[...]
    hn = _silu(jnp.dot(h, wa_ref[...], preferred_element_type=jnp.float32)
               + jnp.dot(agg, wb_ref[...], preferred_element_type=jnp.float32)
               + nb1_ref[...])
    hn = jnp.dot(hn, w2_ref[...], preferred_element_type=jnp.float32) + nb2_ref[...]
    h_out_ref[...] = h + hn
    axs = ax0_ref[...] + ax1_ref[...]
    cnt = jnp.maximum(axs[:, 3:4], 1.0)
    xup = jnp.concatenate(
        [axs / cnt, jnp.zeros((axs.shape[0], H - 16), jnp.float32)], axis=1)
    x_out_ref[...] = xp_ref[...] + xup


def _node_pipeline(h, am0, am1, ax0, ax1, xp, Wa, Wb, nb1, W2, nb2):
    grid = (N // NT,)
    full = lambda shape: pl.BlockSpec(shape, lambda i: (0, 0))
    return pl.pallas_call(
        _node_body,
        grid=grid,
        in_specs=[
            pl.BlockSpec((NT, H), lambda i: (i, 0)),
            pl.BlockSpec((NT, H), lambda i: (i, 0)),
            pl.BlockSpec((NT, H), lambda i: (i, 0)),
            pl.BlockSpec((NT, 16), lambda i: (i, 0)),
            pl.BlockSpec((NT, 16), lambda i: (i, 0)),
            pl.BlockSpec((NT, H), lambda i: (i, 0)),
            full((H, H)), full((H, H)), full((1, H)), full((H, D)), full((1, D)),
        ],
        out_specs=[
            pl.BlockSpec((NT, D), lambda i: (i, 0)),
            pl.BlockSpec((NT, H), lambda i: (i, 0)),
        ],
        out_shape=[
            jax.ShapeDtypeStruct((N, D), jnp.float32),
            jax.ShapeDtypeStruct((N, H), jnp.float32),
        ],
    )(h, am0, am1, ax0, ax1, xp, Wa, Wb, nb1, W2, nb2)


def _proj_body(h_ref, xp_ref, w_ref, ax_ref, bx_ref):
    h = h_ref[...]
    xp = xp_ref[...]
    ax_ref[:, 0:H] = jnp.dot(h, w_ref[0:D, :], preferred_element_type=jnp.float32)
    ax_ref[:, H:2 * H] = xp
    bx_ref[:, 0:H] = jnp.dot(h, w_ref[D:2 * D, :],
                             preferred_element_type=jnp.float32)
    bx_ref[:, H:2 * H] = xp


def _proj_pipeline(h, xp, W1d):
    grid = (N // NT,)
    return pl.pallas_call(
        _proj_body,
        grid=grid,
        in_specs=[
            pl.BlockSpec((NT, D), lambda i: (i, 0)),
            pl.BlockSpec((NT, H), lambda i: (i, 0)),
            pl.BlockSpec((2 * D, H), lambda i: (0, 0)),
        ],
        out_specs=[
            pl.BlockSpec((NT, 2 * H), lambda i: (i, 0)),
            pl.BlockSpec((NT, 2 * H), lambda i: (i, 0)),
        ],
        out_shape=[
            jax.ShapeDtypeStruct((N, 2 * H), jnp.float32),
            jax.ShapeDtypeStruct((N, 2 * H), jnp.float32),
        ],
    )(h, xp, W1d)


def _fc_body(h_ref, w_ref, b_ref, y_ref):
    y_ref[...] = jnp.dot(h_ref[...], w_ref[...],
                         preferred_element_type=jnp.float32) + b_ref[...]


def _fc_pipeline(h, fc_W, fc_b):
    grid = (N // NT,)
    return pl.pallas_call(
        _fc_body,
        grid=grid,
        in_specs=[
            pl.BlockSpec((NT, D), lambda i: (i, 0)),
            pl.BlockSpec((D, 128), lambda i: (0, 0)),
            pl.BlockSpec((1, 128), lambda i: (0, 0)),
        ],
        out_specs=pl.BlockSpec((NT, 128), lambda i: (i, 0)),
        out_shape=jax.ShapeDtypeStruct((N, 128), jnp.float32),
    )(h, fc_W, fc_b[None, :])


def kernel(h, x, edge_index, edge_W1, edge_b1, edge_W2, edge_b2,
           coord_W1, coord_b1, coord_W2, coord_b2,
           node_W1, node_b1, node_W2, node_b2, fc_W, fc_b):
    src2 = edge_index[0].reshape(ROWS, CH)
    dst2 = edge_index[1].reshape(ROWS, CH)

    xp = jnp.pad(x, ((0, 0), (0, H - 3)))

    for l in range(L):
        AX, BX = _proj_pipeline(h, xp, edge_W1[l])
        wr = edge_W1[l, 2 * D:2 * D + 1, :]

        GXA, GXB = _sc_gather(AX, BX, dst2, src2)

        m, t = _edge_pipeline(
            GXA, GXB, wr, edge_b1[l][None, :], edge_W2[l],
            edge_b2[l][None, :], coord_W1[l], coord_b1[l][None, :],
            coord_W2[l], coord_b2[l][None, :])

        am = _sc_scatter(m, dst2)
        dst = edge_index[1]
        ax0 = jax.ops.segment_sum(t, dst, num_segments=N)
        ax1 = jnp.zeros_like(ax0)

        h, xp = _node_pipeline(
            h, am[0][0, :N], am[0][1, :N], ax0, ax1, xp,
            node_W1[l, 0:D, :], node_W1[l, D:, :], node_b1[l][None, :],
            node_W2[l], node_b2[l][None, :])

    return _fc_pipeline(h, fc_W, fc_b)
